# baseline (speedup 1.0000x reference)
import functools

import jax
import jax.numpy as jnp
from jax import lax
from jax.experimental import pallas as pl
from jax.experimental.pallas import tpu as pltpu
from jax.experimental.pallas import tpu_sc as plsc

N_PAD = 10240
NB = 64
NBLK = N_PAD // NB
EC = 64
NW = 32


def _dense_body(x_ref, w_ref, ms_ref, md_ref, h_ref, as_ref, ad_ref):
    h = x_ref[...] @ w_ref[...]
    h_ref[...] = h
    as_ref[...] = h @ ms_ref[...]
    ad_ref[...] = h @ md_ref[...]


def _dense_stage(xp, W, ms, md):
    K = xp.shape[1]
    HC = W.shape[1]
    RB = 512
    return pl.pallas_call(
        _dense_body,
        grid=(N_PAD // RB,),
        in_specs=[
            pl.BlockSpec((RB, K), lambda i: (i, 0)),
            pl.BlockSpec((K, HC), lambda i: (0, 0)),
            pl.BlockSpec((HC, 128), lambda i: (0, 0)),
            pl.BlockSpec((HC, 128), lambda i: (0, 0)),
        ],
        out_specs=[
            pl.BlockSpec((RB, HC), lambda i: (i, 0)),
            pl.BlockSpec((RB, 128), lambda i: (i, 0)),
            pl.BlockSpec((RB, 128), lambda i: (i, 0)),
        ],
        out_shape=[
            jax.ShapeDtypeStruct((N_PAD, HC), jnp.float32),
            jax.ShapeDtypeStruct((N_PAD, 128), jnp.float32),
            jax.ShapeDtypeStruct((N_PAD, 128), jnp.float32),
        ],
    )(xp, W, ms, md)


def _attn_mat(a, H, C):
    HC = H * C
    rows = jnp.arange(HC)
    return jnp.zeros((HC, 128), jnp.float32).at[rows, rows // C].set(a.reshape(HC))


def _final_body(x1_ref, x2_ref, x3_ref, w1_ref, w2_ref, w3_ref, bf_ref, o_ref):
    acc = x1_ref[...] @ w1_ref[...]
    acc += x2_ref[...] @ w2_ref[...]
    acc += x3_ref[...] @ w3_ref[...]
    o_ref[...] = jax.nn.sigmoid(acc + bf_ref[0, 0])


def _final_stage(x1, x2, x3, Wf, bf):
    RB = 1024
    return pl.pallas_call(
        _final_body,
        grid=(N_PAD // RB,),
        in_specs=[
            pl.BlockSpec((RB, 256), lambda i: (i, 0)),
            pl.BlockSpec((RB, 256), lambda i: (i, 0)),
            pl.BlockSpec((RB, 768), lambda i: (i, 0)),
            pl.BlockSpec((256, 1), lambda i: (0, 0)),
            pl.BlockSpec((256, 1), lambda i: (0, 0)),
            pl.BlockSpec((768, 1), lambda i: (0, 0)),
            pl.BlockSpec((1, 1), lambda i: (0, 0)),
        ],
        out_specs=pl.BlockSpec((RB, 1), lambda i: (i, 0)),
        out_shape=jax.ShapeDtypeStruct((N_PAD, 1), jnp.float32),
    )(x1, x2, x3, Wf[:256], Wf[256:512], Wf[512:], bf.reshape(1, 1))


def _make_sc_agg(H, C, NB, EC, UE):
    HC = H * C
    NBLK = N_PAD // NB
    mesh = plsc.VectorSubcoreMesh(core_axis_name="c", subcore_axis_name="s")

    @functools.partial(
        pl.kernel,
        out_type=jax.ShapeDtypeStruct((N_PAD, HC), jnp.float32),
        mesh=mesh,
        scratch_types=[
            pltpu.VMEM((NBLK + 16,), jnp.int32),
            pltpu.VMEM((HC,), jnp.float32),
            pltpu.VMEM((EC,), jnp.int32),
            pltpu.VMEM((EC,), jnp.int32),
            pltpu.VMEM((EC,), jnp.int32),
            pltpu.VMEM((EC,), jnp.int32),
            pltpu.VMEM((2, EC, 128), jnp.float32),
            pltpu.VMEM((2, EC, HC), jnp.float32),
            pltpu.VMEM((NB, 128), jnp.float32),
            pltpu.VMEM((NB, HC), jnp.float32),
            pltpu.VMEM((NB, 16), jnp.float32),
            pltpu.SemaphoreType.DMA,
            pltpu.SemaphoreType.DMA,
            pltpu.SemaphoreType.DMA,
            pltpu.SemaphoreType.DMA,
        ],
        compiler_params=pltpu.CompilerParams(needs_layout_passes=False),
    )
    def sc_agg(h_hbm, as_hbm, ad_hbm, src_hbm, dst_hbm, ptr_hbm, bias_hbm,
               out_hbm, ptr_v, bias_v, src0_v, src1_v, dst0_v, dst1_v,
               asg_v, rows_v, adb_v, acc_v, sacc_v, sr0, sr1, sa0, sa1):
        wid = lax.axis_index("s") * 2 + lax.axis_index("c")
        pltpu.sync_copy(ptr_hbm, ptr_v)
        pltpu.sync_copy(bias_hbm, bias_v)
        lanes = lax.iota(jnp.int32, 16)
        zero16 = jnp.zeros((16,), jnp.float32)
        srs = (sr0, sr1)
        sas = (sa0, sa1)
        srcs = (src0_v, src1_v)
        dsts = (dst0_v, dst1_v)

        def blk_body(bi, carry):
            b = bi * NW + wid
            base = b * NB
            p0 = jnp.max(plsc.load_gather(ptr_v, [jnp.full((16,), b, jnp.int32)]))
            p1 = jnp.max(plsc.load_gather(ptr_v, [jnp.full((16,), b + 1, jnp.int32)]))
            pltpu.sync_copy(ad_hbm.at[pl.ds(base, NB)], adb_v)

            def zero_row(r, c):
                for g in range(HC // 16):
                    acc_v[r, g * 16:(g + 1) * 16] = zero16
                sacc_v[r, :] = zero16
                return c
            lax.fori_loop(0, NB, zero_row, 0)

            c0 = (p0 // EC) * EC
            nch = (p1 - c0 + EC - 1) // EC

            def issue(j, buf):
                ce = c0 + j * EC
                pltpu.sync_copy(src_hbm.at[pl.ds(ce, EC)], srcs[buf])
                pltpu.sync_copy(dst_hbm.at[pl.ds(ce, EC)], dsts[buf])
                pltpu.async_copy(h_hbm.at[srcs[buf]], rows_v.at[buf], srs[buf])
                pltpu.async_copy(as_hbm.at[srcs[buf]], asg_v.at[buf], sas[buf])

            def wait(buf):
                pltpu.make_async_copy(h_hbm.at[srcs[buf]], rows_v.at[buf],
                                      srs[buf]).wait()
                pltpu.make_async_copy(as_hbm.at[srcs[buf]], asg_v.at[buf],
                                      sas[buf]).wait()

            def compute(ci, buf):
                ce = c0 + ci * EC

                def group_body(k, cc):
                    e16 = lanes + k * 16
                    eg = e16 + ce
                    m = (eg >= p0) & (eg < p1)
                    dloc = dsts[buf][pl.ds(k * 16, 16)] - base
                    dloc = jnp.clip(dloc, 0, NB - 1)
                    ws = []
                    for h in range(H):
                        h16 = jnp.full((16,), h, jnp.int32)
                        zs = plsc.load_gather(asg_v.at[buf], [e16, h16])
                        zd = plsc.load_gather(adb_v, [dloc, h16])
                        z = zs + zd
                        w = jnp.exp(jnp.maximum(z, 0.2 * z))
                        w = jnp.where(m, w, 0.0)
                        plsc.addupdate_scatter(sacc_v, [dloc, h16], w, mask=m)
                        ws.append(w)

                    def edge_group(eo, wcarry):
                        e0 = eo * UE
                        dls = [jnp.max(jnp.where(lanes == e0 + e, dloc, 0))
                               for e in range(UE)]
                        wbs = [[wcarry[h].at[
                                    (jnp.full((16,), e0, jnp.int32) + e)].get(
                                        mode="promise_in_bounds")
                                for h in range(H)] for e in range(UE)]
                        for e in range(UE):
                            le = k * 16 + e0 + e
                            dl = dls[e]
                            for h in range(H):
                                wb = wbs[e][h]
                                for g in range(C // 16):
                                    col = h * C + g * 16
                                    acc_v[dl, col:col + 16] = (
                                        acc_v[dl, col:col + 16]
                                        + wb * rows_v[buf, le, col:col + 16])
                        return wcarry
                    lax.fori_loop(0, 16 // UE, edge_group, tuple(ws))
                    return cc
                lax.fori_loop(0, EC // 16, group_body, 0)

            @pl.when(nch > 0)
            def _():
                issue(0, 0)

            def pipe_body(ci2, carry):
                for half in range(2):
                    ci = ci2 * 2 + half

                    @pl.when(ci < nch)
                    def _():
                        @pl.when(ci + 1 < nch)
                        def _():
                            issue(ci + 1, 1 - half)
                        wait(half)
                        compute(ci, half)
                return carry
            lax.fori_loop(0, (nch + 1) // 2, pipe_body, 0)

            def flush_row(r, c):
                r16 = jnp.full((16,), r, jnp.int32)
                for h in range(H):
                    sv = plsc.load_gather(sacc_v, [r16, jnp.full((16,), h, jnp.int32)])
                    for g in range(C // 16):
                        col = h * C + g * 16
                        v = acc_v[r, col:col + 16] / sv + bias_v[col:col + 16]
                        acc_v[r, col:col + 16] = jnp.maximum(v, 0.0)
                return c
            lax.fori_loop(0, NB, flush_row, 0)
            pltpu.sync_copy(acc_v, out_hbm.at[pl.ds(base, NB)])
            return carry
        lax.fori_loop(0, NBLK // NW, blk_body, 0)

    return sc_agg


_sc_agg_8_32 = _make_sc_agg(8, 32, 64, 64, 16)
_sc_agg_12_64 = _make_sc_agg(12, 64, 32, 32, 4)


def kernel(x, edge_index, JetRawPt, W1, a1s, a1d, b1, W2, a2s, a2d, b2, W3,
           a3s, a3d, b3, Wf, bf):
    N = x.shape[0]
    ei = edge_index.astype(jnp.int32)
    loopidx = jnp.arange(N, dtype=jnp.int32)
    src = jnp.concatenate([ei[0], loopidx])
    dst = jnp.concatenate([ei[1], loopidx])
    order = jnp.argsort(dst)
    src_s = src[order]
    dst_s = dst[order]
    E = src_s.shape[0]
    E_PAD = ((E + EC - 1) // EC) * EC

    def _ptr(NB):
        nblk = N_PAD // NB
        bounds = jnp.arange(nblk + 1, dtype=jnp.int32) * NB
        p = jnp.searchsorted(dst_s, bounds).astype(jnp.int32)
        return jnp.concatenate([p, jnp.full((15,), E, jnp.int32)])
    ptr64 = _ptr(64)
    ptr32 = _ptr(32)

    src_p = jnp.concatenate([src_s, jnp.zeros((E_PAD - E,), jnp.int32)])
    dst_p = jnp.concatenate([dst_s, jnp.zeros((E_PAD - E,), jnp.int32)])

    xp = jnp.pad(x, ((0, N_PAD - N), (0, 0)))

    h1, as1, ad1 = _dense_stage(xp, W1, _attn_mat(a1s, 8, 32), _attn_mat(a1d, 8, 32))
    x1 = _sc_agg_8_32(h1, as1, ad1, src_p, dst_p, ptr64, b1)
    h2, as2, ad2 = _dense_stage(x1, W2, _attn_mat(a2s, 8, 32), _attn_mat(a2d, 8, 32))
    x2 = _sc_agg_8_32(h2, as2, ad2, src_p, dst_p, ptr64, b2)
    h3, as3, ad3 = _dense_stage(x2, W3, _attn_mat(a3s, 12, 64), _attn_mat(a3d, 12, 64))
    x3 = _sc_agg_12_64(h3, as3, ad3, src_p, dst_p, ptr32, b3)

    out = _final_stage(x1, x2, x3, Wf, bf)
    return out[:N]


# speedup vs baseline: 1.2765x; 1.2765x over previous
import functools

import jax
import jax.numpy as jnp
from jax import lax
from jax.experimental import pallas as pl
from jax.experimental.pallas import tpu as pltpu
from jax.experimental.pallas import tpu_sc as plsc

N_PAD = 10240
NB = 64
NBLK = N_PAD // NB
EC = 64
NW = 32


def _dense_body(x_ref, w_ref, ms_ref, md_ref, h_ref, as_ref, ad_ref):
    h = x_ref[...] @ w_ref[...]
    h_ref[...] = h
    as_ref[...] = h @ ms_ref[...]
    ad_ref[...] = h @ md_ref[...]


def _dense_stage(xp, W, ms, md):
    K = xp.shape[1]
    HC = W.shape[1]
    RB = 512
    return pl.pallas_call(
        _dense_body,
        grid=(N_PAD // RB,),
        in_specs=[
            pl.BlockSpec((RB, K), lambda i: (i, 0)),
            pl.BlockSpec((K, HC), lambda i: (0, 0)),
            pl.BlockSpec((HC, 128), lambda i: (0, 0)),
            pl.BlockSpec((HC, 128), lambda i: (0, 0)),
        ],
        out_specs=[
            pl.BlockSpec((RB, HC), lambda i: (i, 0)),
            pl.BlockSpec((RB, 128), lambda i: (i, 0)),
            pl.BlockSpec((RB, 128), lambda i: (i, 0)),
        ],
        out_shape=[
            jax.ShapeDtypeStruct((N_PAD, HC), jnp.float32),
            jax.ShapeDtypeStruct((N_PAD, 128), jnp.float32),
            jax.ShapeDtypeStruct((N_PAD, 128), jnp.float32),
        ],
    )(xp, W, ms, md)


def _attn_mat(a, H, C):
    HC = H * C
    rows = jnp.arange(HC)
    return jnp.zeros((HC, 128), jnp.float32).at[rows, rows // C].set(a.reshape(HC))


def _final_body(x1_ref, x2_ref, x3_ref, w1_ref, w2_ref, w3_ref, bf_ref, o_ref):
    acc = x1_ref[...] @ w1_ref[...]
    acc += x2_ref[...] @ w2_ref[...]
    acc += x3_ref[...] @ w3_ref[...]
    o_ref[...] = jax.nn.sigmoid(acc + bf_ref[0, 0])


def _final_stage(x1, x2, x3, Wf, bf):
    RB = 1024
    return pl.pallas_call(
        _final_body,
        grid=(N_PAD // RB,),
        in_specs=[
            pl.BlockSpec((RB, 256), lambda i: (i, 0)),
            pl.BlockSpec((RB, 256), lambda i: (i, 0)),
            pl.BlockSpec((RB, 768), lambda i: (i, 0)),
            pl.BlockSpec((256, 1), lambda i: (0, 0)),
            pl.BlockSpec((256, 1), lambda i: (0, 0)),
            pl.BlockSpec((768, 1), lambda i: (0, 0)),
            pl.BlockSpec((1, 1), lambda i: (0, 0)),
        ],
        out_specs=pl.BlockSpec((RB, 1), lambda i: (i, 0)),
        out_shape=jax.ShapeDtypeStruct((N_PAD, 1), jnp.float32),
    )(x1, x2, x3, Wf[:256], Wf[256:512], Wf[512:], bf.reshape(1, 1))


def _make_sc_agg(H, C, NB, EC):
    HC = H * C
    NBLK = N_PAD // NB
    mesh = plsc.VectorSubcoreMesh(core_axis_name="c", subcore_axis_name="s")

    @functools.partial(
        pl.kernel,
        out_type=jax.ShapeDtypeStruct((N_PAD, HC), jnp.float32),
        mesh=mesh,
        scratch_types=[
            pltpu.VMEM((NBLK + 16,), jnp.int32),
            pltpu.VMEM((HC,), jnp.float32),
            pltpu.VMEM((EC,), jnp.int32),
            pltpu.VMEM((EC,), jnp.int32),
            pltpu.VMEM((EC,), jnp.int32),
            pltpu.VMEM((EC,), jnp.int32),
            pltpu.VMEM((2, EC, 128), jnp.float32),
            pltpu.VMEM((2, EC, HC), jnp.float32),
            pltpu.VMEM((NB, 128), jnp.float32),
            pltpu.VMEM((NB, HC), jnp.float32),
            pltpu.VMEM((NB, 16), jnp.float32),
            pltpu.SemaphoreType.DMA,
            pltpu.SemaphoreType.DMA,
            pltpu.SemaphoreType.DMA,
            pltpu.SemaphoreType.DMA,
        ],
        compiler_params=pltpu.CompilerParams(needs_layout_passes=False),
    )
    def sc_agg(h_hbm, as_hbm, ad_hbm, src_hbm, dst_hbm, ptr_hbm, bias_hbm,
               out_hbm, ptr_v, bias_v, src0_v, src1_v, dst0_v, dst1_v,
               asg_v, rows_v, adb_v, acc_v, sacc_v, sr0, sr1, sa0, sa1):
        wid = lax.axis_index("s") * 2 + lax.axis_index("c")
        pltpu.sync_copy(ptr_hbm, ptr_v)
        pltpu.sync_copy(bias_hbm, bias_v)
        lanes = lax.iota(jnp.int32, 16)
        zero16 = jnp.zeros((16,), jnp.float32)
        srs = (sr0, sr1)
        sas = (sa0, sa1)
        srcs = (src0_v, src1_v)
        dsts = (dst0_v, dst1_v)

        def blk_body(bi, carry):
            b = bi * NW + wid
            base = b * NB
            p0 = jnp.max(plsc.load_gather(ptr_v, [jnp.full((16,), b, jnp.int32)]))
            p1 = jnp.max(plsc.load_gather(ptr_v, [jnp.full((16,), b + 1, jnp.int32)]))
            pltpu.sync_copy(ad_hbm.at[pl.ds(base, NB)], adb_v)

            def zero_row(r, c):
                for g in range(HC // 16):
                    acc_v[r, g * 16:(g + 1) * 16] = zero16
                sacc_v[r, :] = zero16
                return c
            lax.fori_loop(0, NB, zero_row, 0)

            c0 = (p0 // EC) * EC
            nch = (p1 - c0 + EC - 1) // EC

            def issue(j, buf):
                ce = c0 + j * EC
                pltpu.sync_copy(src_hbm.at[pl.ds(ce, EC)], srcs[buf])
                pltpu.sync_copy(dst_hbm.at[pl.ds(ce, EC)], dsts[buf])
                pltpu.async_copy(h_hbm.at[srcs[buf]], rows_v.at[buf], srs[buf])
                pltpu.async_copy(as_hbm.at[srcs[buf]], asg_v.at[buf], sas[buf])

            def wait(buf):
                pltpu.make_async_copy(h_hbm.at[srcs[buf]], rows_v.at[buf],
                                      srs[buf]).wait()
                pltpu.make_async_copy(as_hbm.at[srcs[buf]], asg_v.at[buf],
                                      sas[buf]).wait()

            def compute(ci, buf):
                ce = c0 + ci * EC

                def group_body(k, cc):
                    e16 = lanes + k * 16
                    eg = e16 + ce
                    m = (eg >= p0) & (eg < p1)
                    dloc = dsts[buf][pl.ds(k * 16, 16)] - base
                    dloc = jnp.clip(dloc, 0, NB - 1)
                    ws = []
                    for h in range(H):
                        h16 = jnp.full((16,), h, jnp.int32)
                        zs = plsc.load_gather(asg_v.at[buf], [e16, h16])
                        zd = plsc.load_gather(adb_v, [dloc, h16])
                        z = zs + zd
                        w = jnp.exp(jnp.maximum(z, 0.2 * z))
                        w = jnp.where(m, w, 0.0)
                        plsc.addupdate_scatter(sacc_v, [dloc, h16], w, mask=m)
                        ws.append(w)

                    def edge_body(e, wcarry):
                        le = k * 16 + e
                        dl = jnp.max(jnp.where(lanes == e, dloc, 0))
                        eb = jnp.full((16,), e, jnp.int32)
                        for h in range(H):
                            wb = wcarry[h].at[eb].get(mode="promise_in_bounds")
                            for g in range(C // 16):
                                col = h * C + g * 16
                                acc_v[dl, col:col + 16] = (
                                    acc_v[dl, col:col + 16]
                                    + wb * rows_v[buf, le, col:col + 16])
                        return wcarry
                    lax.fori_loop(0, 16, edge_body, tuple(ws))
                    return cc
                lax.fori_loop(0, EC // 16, group_body, 0)

            @pl.when(nch > 0)
            def _():
                issue(0, 0)

            def pipe_body(ci2, carry):
                for half in range(2):
                    ci = ci2 * 2 + half

                    @pl.when(ci < nch)
                    def _():
                        @pl.when(ci + 1 < nch)
                        def _():
                            issue(ci + 1, 1 - half)
                        wait(half)
                        compute(ci, half)
                return carry
            lax.fori_loop(0, (nch + 1) // 2, pipe_body, 0)

            def flush_row(r, c):
                r16 = jnp.full((16,), r, jnp.int32)
                for h in range(H):
                    sv = plsc.load_gather(sacc_v, [r16, jnp.full((16,), h, jnp.int32)])
                    for g in range(C // 16):
                        col = h * C + g * 16
                        v = acc_v[r, col:col + 16] / sv + bias_v[col:col + 16]
                        acc_v[r, col:col + 16] = jnp.maximum(v, 0.0)
                return c
            lax.fori_loop(0, NB, flush_row, 0)
            pltpu.sync_copy(acc_v, out_hbm.at[pl.ds(base, NB)])
            return carry
        lax.fori_loop(0, NBLK // NW, blk_body, 0)

    return sc_agg


_sc_agg_8_32 = _make_sc_agg(8, 32, 64, 64)
_sc_agg_12_64 = _make_sc_agg(12, 64, 32, 48)


def kernel(x, edge_index, JetRawPt, W1, a1s, a1d, b1, W2, a2s, a2d, b2, W3,
           a3s, a3d, b3, Wf, bf):
    N = x.shape[0]
    ei = edge_index.astype(jnp.int32)
    loopidx = jnp.arange(N, dtype=jnp.int32)
    src = jnp.concatenate([ei[0], loopidx])
    dst = jnp.concatenate([ei[1], loopidx])
    order = jnp.argsort(dst)
    src_s = src[order]
    dst_s = dst[order]
    E = src_s.shape[0]
    E_PAD = ((E + EC - 1) // EC) * EC

    def _ptr(NB):
        nblk = N_PAD // NB
        bounds = jnp.arange(nblk + 1, dtype=jnp.int32) * NB
        p = jnp.searchsorted(dst_s, bounds).astype(jnp.int32)
        return jnp.concatenate([p, jnp.full((15,), E, jnp.int32)])
    ptr64 = _ptr(64)
    ptr32 = _ptr(32)

    src_p = jnp.concatenate([src_s, jnp.zeros((E_PAD - E,), jnp.int32)])
    dst_p = jnp.concatenate([dst_s, jnp.zeros((E_PAD - E,), jnp.int32)])

    xp = jnp.pad(x, ((0, N_PAD - N), (0, 0)))

    h1, as1, ad1 = _dense_stage(xp, W1, _attn_mat(a1s, 8, 32), _attn_mat(a1d, 8, 32))
    x1 = _sc_agg_8_32(h1, as1, ad1, src_p, dst_p, ptr64, b1)
    h2, as2, ad2 = _dense_stage(x1, W2, _attn_mat(a2s, 8, 32), _attn_mat(a2d, 8, 32))
    x2 = _sc_agg_8_32(h2, as2, ad2, src_p, dst_p, ptr64, b2)
    h3, as3, ad3 = _dense_stage(x2, W3, _attn_mat(a3s, 12, 64), _attn_mat(a3d, 12, 64))
    x3 = _sc_agg_12_64(h3, as3, ad3, src_p, dst_p, ptr32, b3)

    out = _final_stage(x1, x2, x3, Wf, bf)
    return out[:N]
